# baseline (speedup 1.0000x reference)
import functools

import jax
import jax.numpy as jnp
from jax import lax
from jax.experimental import pallas as pl
from jax.experimental.pallas import tpu as pltpu
from jax.experimental.pallas import tpu_sc as plsc

_OFFSET = 0.001

_INFO = plsc.get_sparse_core_info()
_NC, _NS, _L = _INFO.num_cores, _INFO.num_subcores, _INFO.num_lanes
_NW = _NC * _NS

_TOTAL = 262144 * 256
_PER_W = _TOTAL // _NW
_CHUNK = 16384
_NCHUNK = _PER_W // _CHUNK


def _sc_body(x_hbm, o_hbm, in_v, out_v):
    wid = lax.axis_index("s") * _NC + lax.axis_index("c")
    base = wid * _PER_W

    def chunk(i, carry):
        off = base + i * _CHUNK
        pltpu.sync_copy(x_hbm.at[pl.ds(off, _CHUNK)], in_v)

        def vec(j, c):
            x = in_v[pl.ds(j * _L, _L)]
            out_v[pl.ds(j * _L, _L)] = 1.0 / (jnp.abs(x) + _OFFSET)
            return c

        lax.fori_loop(0, _CHUNK // _L, vec, 0)
        pltpu.sync_copy(out_v, o_hbm.at[pl.ds(off, _CHUNK)])
        return carry

    lax.fori_loop(0, _NCHUNK, chunk, 0)


_sc_call = functools.partial(
    pl.kernel,
    out_type=jax.ShapeDtypeStruct((_TOTAL,), jnp.float32),
    mesh=plsc.VectorSubcoreMesh(core_axis_name="c", subcore_axis_name="s"),
    scratch_types=[
        pltpu.VMEM((_CHUNK,), jnp.float32),
        pltpu.VMEM((_CHUNK,), jnp.float32),
    ],
)(_sc_body)


def kernel(xyz):
    n, d = xyz.shape
    flat = xyz.reshape(-1)
    out = _sc_call(flat)
    return out.reshape(n, d)


# speedup vs baseline: 1.8009x; 1.8009x over previous
import functools

import jax
import jax.numpy as jnp
from jax import lax
from jax.experimental import pallas as pl
from jax.experimental.pallas import tpu as pltpu
from jax.experimental.pallas import tpu_sc as plsc

_OFFSET = 0.001

_INFO = plsc.get_sparse_core_info()
_NC, _NS, _L = _INFO.num_cores, _INFO.num_subcores, _INFO.num_lanes
_NW = _NC * _NS

_TOTAL = 262144 * 256
_PER_W = _TOTAL // _NW
_CHUNK = 16384
_NCHUNK = _PER_W // _CHUNK
_NBUF = 2


def _sc_body(x_hbm, o_hbm, in_v, out_v, gsem, ssem):
    wid = lax.axis_index("s") * _NC + lax.axis_index("c")
    base = wid * _PER_W

    def gather(i, b):
        return pltpu.make_async_copy(
            x_hbm.at[pl.ds(base + i * _CHUNK, _CHUNK)], in_v.at[b], gsem.at[b]
        )

    def scatter(i, b):
        return pltpu.make_async_copy(
            out_v.at[b], o_hbm.at[pl.ds(base + i * _CHUNK, _CHUNK)], ssem.at[b]
        )

    for b in range(_NBUF):
        gather(b, b).start()

    def outer(k, carry):
        i0 = k * _NBUF
        for b in range(_NBUF):
            i = i0 + b
            gather(i, b).wait()

            @pl.when(k > 0)
            def _():
                scatter(i - _NBUF, b).wait()

            @plsc.parallel_loop(0, _CHUNK // _L, unroll=8)
            def _(j):
                x = in_v[b, pl.ds(j * _L, _L)]
                out_v[b, pl.ds(j * _L, _L)] = 1.0 / (jnp.abs(x) + _OFFSET)

            scatter(i, b).start()

            @pl.when(i + _NBUF < _NCHUNK)
            def _():
                gather(i + _NBUF, b).start()

        return carry

    lax.fori_loop(0, _NCHUNK // _NBUF, outer, 0)

    for b in range(_NBUF):
        scatter(_NCHUNK - _NBUF + b, b).wait()


_sc_call = functools.partial(
    pl.kernel,
    out_type=jax.ShapeDtypeStruct((_TOTAL,), jnp.float32),
    mesh=plsc.VectorSubcoreMesh(core_axis_name="c", subcore_axis_name="s"),
    scratch_types=[
        pltpu.VMEM((_NBUF, _CHUNK), jnp.float32),
        pltpu.VMEM((_NBUF, _CHUNK), jnp.float32),
        pltpu.SemaphoreType.DMA((_NBUF,)),
        pltpu.SemaphoreType.DMA((_NBUF,)),
    ],
)(_sc_body)


def kernel(xyz):
    n, d = xyz.shape
    flat = xyz.reshape(-1)
    out = _sc_call(flat)
    return out.reshape(n, d)


# speedup vs baseline: 6.5657x; 3.6457x over previous
import functools

import jax
import jax.numpy as jnp
from jax import lax
from jax.experimental import pallas as pl
from jax.experimental.pallas import tpu as pltpu
from jax.experimental.pallas import tpu_sc as plsc

_OFFSET = 0.001

_INFO = plsc.get_sparse_core_info()
_NC, _NS, _L = _INFO.num_cores, _INFO.num_subcores, _INFO.num_lanes
_NW = _NC * _NS

_N = 262144
_D = 256
_ROWS_W = _N // _NW
_CROWS = 64
_NCHUNK = _ROWS_W // _CROWS
_NBUF = 2


def _sc_body(x_hbm, o_hbm, in_v, out_v, gsem, ssem):
    wid = lax.axis_index("s") * _NC + lax.axis_index("c")
    base = wid * _ROWS_W

    def gather(i, b):
        return pltpu.make_async_copy(
            x_hbm.at[pl.ds(base + i * _CROWS, _CROWS)], in_v.at[b], gsem.at[b]
        )

    def scatter(i, b):
        return pltpu.make_async_copy(
            out_v.at[b], o_hbm.at[pl.ds(base + i * _CROWS, _CROWS)], ssem.at[b]
        )

    for b in range(_NBUF):
        gather(b, b).start()

    def outer(k, carry):
        i0 = k * _NBUF
        for b in range(_NBUF):
            i = i0 + b
            gather(i, b).wait()

            @pl.when(k > 0)
            def _():
                scatter(i - _NBUF, b).wait()

            @plsc.parallel_loop(0, _CROWS, unroll=2)
            def _(r):
                for c in range(_D // _L):
                    x = in_v[b, r, pl.ds(c * _L, _L)]
                    out_v[b, r, pl.ds(c * _L, _L)] = 1.0 / (jnp.abs(x) + _OFFSET)

            scatter(i, b).start()

            @pl.when(i + _NBUF < _NCHUNK)
            def _():
                gather(i + _NBUF, b).start()

        return carry

    lax.fori_loop(0, _NCHUNK // _NBUF, outer, 0)

    for b in range(_NBUF):
        scatter(_NCHUNK - _NBUF + b, b).wait()


_sc_call = functools.partial(
    pl.kernel,
    out_type=jax.ShapeDtypeStruct((_N, _D), jnp.float32),
    mesh=plsc.VectorSubcoreMesh(core_axis_name="c", subcore_axis_name="s"),
    scratch_types=[
        pltpu.VMEM((_NBUF, _CROWS, _D), jnp.float32),
        pltpu.VMEM((_NBUF, _CROWS, _D), jnp.float32),
        pltpu.SemaphoreType.DMA((_NBUF,)),
        pltpu.SemaphoreType.DMA((_NBUF,)),
    ],
    compiler_params=pltpu.CompilerParams(use_tc_tiling_on_sc=True),
)(_sc_body)


def kernel(xyz):
    return _sc_call(xyz)


# speedup vs baseline: 6.5883x; 1.0034x over previous
import functools

import jax
import jax.numpy as jnp
from jax import lax
from jax.experimental import pallas as pl
from jax.experimental.pallas import tpu as pltpu
from jax.experimental.pallas import tpu_sc as plsc

_OFFSET = 0.001

_INFO = plsc.get_sparse_core_info()
_NC, _NS, _L = _INFO.num_cores, _INFO.num_subcores, _INFO.num_lanes
_NW = _NC * _NS

_N = 262144
_D = 256
_ROWS_W = _N // _NW
_CROWS = 64
_NCHUNK = _ROWS_W // _CROWS
_NBUF = 4


def _sc_body(x_hbm, o_hbm, buf, gsem, ssem):
    wid = lax.axis_index("s") * _NC + lax.axis_index("c")
    base = wid * _ROWS_W

    def gather(i, b):
        return pltpu.make_async_copy(
            x_hbm.at[pl.ds(base + i * _CROWS, _CROWS)], buf.at[b], gsem.at[b]
        )

    def scatter(i, b):
        return pltpu.make_async_copy(
            buf.at[b], o_hbm.at[pl.ds(base + i * _CROWS, _CROWS)], ssem.at[b]
        )

    for b in range(_NBUF - 1):
        gather(b, b).start()

    def outer(k, carry):
        i0 = k * _NBUF
        for j in range(_NBUF):
            i = i0 + j
            gather(i, j).wait()

            @plsc.parallel_loop(0, _CROWS, unroll=2)
            def _(r):
                for c in range(_D // _L):
                    x = buf[j, r, pl.ds(c * _L, _L)]
                    buf[j, r, pl.ds(c * _L, _L)] = 1.0 / (jnp.abs(x) + _OFFSET)

            scatter(i, j).start()
            nb = (j + _NBUF - 1) % _NBUF

            @pl.when(i + _NBUF - 1 < _NCHUNK)
            def _():
                @pl.when(i >= 1)
                def _():
                    scatter(i - 1, nb).wait()

                gather(i + _NBUF - 1, nb).start()

        return carry

    lax.fori_loop(0, _NCHUNK // _NBUF, outer, 0)

    for j in range(_NBUF):
        scatter(_NCHUNK - _NBUF + j, j).wait()


_sc_call = functools.partial(
    pl.kernel,
    out_type=jax.ShapeDtypeStruct((_N, _D), jnp.float32),
    mesh=plsc.VectorSubcoreMesh(core_axis_name="c", subcore_axis_name="s"),
    scratch_types=[
        pltpu.VMEM((_NBUF, _CROWS, _D), jnp.float32),
        pltpu.SemaphoreType.DMA((_NBUF,)),
        pltpu.SemaphoreType.DMA((_NBUF,)),
    ],
    compiler_params=pltpu.CompilerParams(use_tc_tiling_on_sc=True),
)(_sc_body)


def kernel(xyz):
    return _sc_call(xyz)
